# baseline (speedup 1.0000x reference)
import jax
import jax.numpy as jnp
from jax.experimental import pallas as pl
from jax.experimental.pallas import tpu as pltpu

D_MODEL = 4096
NUM_EXPERTS = 64
TOP_K = 8
N_TOKENS = 16384
BLK = 1024
GRID = N_TOKENS // BLK
CH = 128


def _router_kernel(u_ref, e_ref, bias_ref, topk_i_ref, topk_s_ref,
                   scores_ref, aux_ref, psum_ref):
    i = pl.program_id(0)
    logits = jnp.dot(u_ref[...], e_ref[...],
                     preferred_element_type=jnp.float32) + bias_ref[...]
    scores = jax.nn.sigmoid(logits)
    scores_ref[...] = scores

    @pl.when(i == 0)
    def _init():
        psum_ref[...] = jnp.zeros_like(psum_ref)

    iota_f = jax.lax.broadcasted_iota(jnp.int32, (CH, NUM_EXPERTS),
                                      1).astype(jnp.float32)

    def body(c, acc):
        sc = scores_ref[pl.ds(c * CH, CH), :]
        e = jnp.exp(sc)
        probs = e / jnp.sum(e, axis=1, keepdims=True)
        acc = acc + jnp.sum(probs, axis=0).reshape(1, NUM_EXPERTS)

        x = sc
        vals = []
        fidxs = []
        for _ in range(TOP_K):
            mx = jnp.max(x, axis=1, keepdims=True)
            idx = jnp.min(jnp.where(x == mx, iota_f,
                                    jnp.float32(NUM_EXPERTS)),
                          axis=1, keepdims=True)
            vals.append(mx)
            fidxs.append(idx)
            x = jnp.where(iota_f == idx, -jnp.inf, x)
        topk_s_ref[pl.ds(c * CH, CH), :] = jnp.concatenate(vals, axis=1)
        topk_i_ref[pl.ds(c * CH, CH), :] = jnp.concatenate(
            fidxs, axis=1).astype(jnp.int32)
        return acc

    acc = jax.lax.fori_loop(0, BLK // CH, body,
                            jnp.zeros((1, NUM_EXPERTS), jnp.float32))
    psum_ref[...] += acc

    @pl.when(i == GRID - 1)
    def _fin():
        mean = psum_ref[...] / N_TOKENS
        aux_ref[...] = (jnp.sum(mean * mean) * NUM_EXPERTS).reshape(1, 1)


def kernel(u, E, bias):
    bias2 = bias.reshape(1, NUM_EXPERTS)
    out_shape = (
        jax.ShapeDtypeStruct((N_TOKENS, TOP_K), jnp.int32),
        jax.ShapeDtypeStruct((N_TOKENS, TOP_K), jnp.float32),
        jax.ShapeDtypeStruct((N_TOKENS, NUM_EXPERTS), jnp.float32),
        jax.ShapeDtypeStruct((1, 1), jnp.float32),
    )
    topk_i, topk_s, scores, aux = pl.pallas_call(
        _router_kernel,
        grid=(GRID,),
        in_specs=[
            pl.BlockSpec((BLK, D_MODEL), lambda i: (i, 0)),
            pl.BlockSpec((D_MODEL, NUM_EXPERTS), lambda i: (0, 0)),
            pl.BlockSpec((1, NUM_EXPERTS), lambda i: (0, 0)),
        ],
        out_specs=(
            pl.BlockSpec((BLK, TOP_K), lambda i: (i, 0)),
            pl.BlockSpec((BLK, TOP_K), lambda i: (i, 0)),
            pl.BlockSpec((BLK, NUM_EXPERTS), lambda i: (i, 0)),
            pl.BlockSpec((1, 1), lambda i: (0, 0)),
        ),
        out_shape=out_shape,
        scratch_shapes=[pltpu.VMEM((1, NUM_EXPERTS), jnp.float32)],
    )(u, E, bias2)
    return topk_i, topk_s, scores, aux[0, 0]


# speedup vs baseline: 1.4564x; 1.4564x over previous
import jax
import jax.numpy as jnp
from jax.experimental import pallas as pl
from jax.experimental.pallas import tpu as pltpu

D_MODEL = 4096
NUM_EXPERTS = 64
TOP_K = 8
N_TOKENS = 16384
BLK = 1024
GRID = N_TOKENS // BLK
CH = 128


def _router_kernel(u_ref, e_ref, bias_ref, topk_i_ref, topk_s_ref,
                   scores_ref, aux_ref, psum_ref):
    i = pl.program_id(0)
    logits = jnp.dot(u_ref[...], e_ref[...],
                     preferred_element_type=jnp.float32) + bias_ref[...]
    scores = jax.nn.sigmoid(logits)
    scores_ref[...] = scores

    @pl.when(i == 0)
    def _init():
        psum_ref[...] = jnp.zeros_like(psum_ref)

    iota_f = jax.lax.broadcasted_iota(jnp.int32, (CH, NUM_EXPERTS),
                                      1).astype(jnp.float32)

    acc = jnp.zeros((1, NUM_EXPERTS), jnp.float32)
    for c in range(BLK // CH):
        sc = scores[c * CH:(c + 1) * CH, :]
        e = jnp.exp(sc)
        probs = e / jnp.sum(e, axis=1, keepdims=True)
        acc = acc + jnp.sum(probs, axis=0).reshape(1, NUM_EXPERTS)

        x = sc
        vals = []
        fidxs = []
        for _ in range(TOP_K):
            mx = jnp.max(x, axis=1, keepdims=True)
            idx = jnp.min(jnp.where(x == mx, iota_f,
                                    jnp.float32(NUM_EXPERTS)),
                          axis=1, keepdims=True)
            vals.append(mx)
            fidxs.append(idx)
            x = jnp.where(iota_f == idx, -jnp.inf, x)
        topk_s_ref[c * CH:(c + 1) * CH, :] = jnp.concatenate(vals, axis=1)
        topk_i_ref[c * CH:(c + 1) * CH, :] = jnp.concatenate(
            fidxs, axis=1).astype(jnp.int32)

    psum_ref[...] += acc

    @pl.when(i == GRID - 1)
    def _fin():
        mean = psum_ref[...] / N_TOKENS
        aux_ref[...] = (jnp.sum(mean * mean) * NUM_EXPERTS).reshape(1, 1)


def kernel(u, E, bias):
    bias2 = bias.reshape(1, NUM_EXPERTS)
    out_shape = (
        jax.ShapeDtypeStruct((N_TOKENS, TOP_K), jnp.int32),
        jax.ShapeDtypeStruct((N_TOKENS, TOP_K), jnp.float32),
        jax.ShapeDtypeStruct((N_TOKENS, NUM_EXPERTS), jnp.float32),
        jax.ShapeDtypeStruct((1, 1), jnp.float32),
    )
    topk_i, topk_s, scores, aux = pl.pallas_call(
        _router_kernel,
        grid=(GRID,),
        in_specs=[
            pl.BlockSpec((BLK, D_MODEL), lambda i: (i, 0)),
            pl.BlockSpec((D_MODEL, NUM_EXPERTS), lambda i: (0, 0)),
            pl.BlockSpec((1, NUM_EXPERTS), lambda i: (0, 0)),
        ],
        out_specs=(
            pl.BlockSpec((BLK, TOP_K), lambda i: (i, 0)),
            pl.BlockSpec((BLK, TOP_K), lambda i: (i, 0)),
            pl.BlockSpec((BLK, NUM_EXPERTS), lambda i: (i, 0)),
            pl.BlockSpec((1, 1), lambda i: (0, 0)),
        ),
        out_shape=out_shape,
        scratch_shapes=[pltpu.VMEM((1, NUM_EXPERTS), jnp.float32)],
    )(u, E, bias2)
    return topk_i, topk_s, scores, aux[0, 0]


# speedup vs baseline: 1.8954x; 1.3014x over previous
import jax
import jax.numpy as jnp
from jax.experimental import pallas as pl
from jax.experimental.pallas import tpu as pltpu

D_MODEL = 4096
NUM_EXPERTS = 64
TOP_K = 8
N_TOKENS = 16384
BLK = 1024
GRID = N_TOKENS // BLK


def _router_kernel(u_ref, e_ref, bias_ref, topk_i_ref, topk_s_ref,
                   scores_ref, aux_ref, psum_ref):
    i = pl.program_id(0)
    logits = jnp.dot(u_ref[...], e_ref[...],
                     preferred_element_type=jnp.float32) + bias_ref[...]
    scores = jax.nn.sigmoid(logits)
    scores_ref[...] = scores

    e = jnp.exp(scores)
    probs = e / jnp.sum(e, axis=1, keepdims=True)
    col = jnp.sum(probs, axis=0).reshape(1, NUM_EXPERTS)

    @pl.when(i == 0)
    def _init():
        psum_ref[...] = jnp.zeros_like(psum_ref)

    psum_ref[...] += col

    iota_f = jax.lax.broadcasted_iota(jnp.int32, scores.shape,
                                      1).astype(jnp.float32)
    x = scores
    vals = []
    fidxs = []
    for _ in range(TOP_K):
        mx = jnp.max(x, axis=1, keepdims=True)
        idx = jnp.min(jnp.where(x == mx, iota_f, jnp.float32(NUM_EXPERTS)),
                      axis=1, keepdims=True)
        vals.append(mx)
        fidxs.append(idx)
        x = jnp.where(iota_f == idx, -jnp.inf, x)
    topk_s_ref[...] = jnp.concatenate(vals, axis=1)
    topk_i_ref[...] = jnp.concatenate(fidxs, axis=1).astype(jnp.int32)

    @pl.when(i == GRID - 1)
    def _fin():
        mean = psum_ref[...] / N_TOKENS
        aux_ref[...] = (jnp.sum(mean * mean) * NUM_EXPERTS).reshape(1, 1)


def kernel(u, E, bias):
    bias2 = bias.reshape(1, NUM_EXPERTS)
    out_shape = (
        jax.ShapeDtypeStruct((N_TOKENS, TOP_K), jnp.int32),
        jax.ShapeDtypeStruct((N_TOKENS, TOP_K), jnp.float32),
        jax.ShapeDtypeStruct((N_TOKENS, NUM_EXPERTS), jnp.float32),
        jax.ShapeDtypeStruct((1, 1), jnp.float32),
    )
    topk_i, topk_s, scores, aux = pl.pallas_call(
        _router_kernel,
        grid=(GRID,),
        in_specs=[
            pl.BlockSpec((BLK, D_MODEL), lambda i: (i, 0)),
            pl.BlockSpec((D_MODEL, NUM_EXPERTS), lambda i: (0, 0)),
            pl.BlockSpec((1, NUM_EXPERTS), lambda i: (0, 0)),
        ],
        out_specs=(
            pl.BlockSpec((BLK, TOP_K), lambda i: (i, 0)),
            pl.BlockSpec((BLK, TOP_K), lambda i: (i, 0)),
            pl.BlockSpec((BLK, NUM_EXPERTS), lambda i: (i, 0)),
            pl.BlockSpec((1, 1), lambda i: (0, 0)),
        ),
        out_shape=out_shape,
        scratch_shapes=[pltpu.VMEM((1, NUM_EXPERTS), jnp.float32)],
    )(u, E, bias2)
    return topk_i, topk_s, scores, aux[0, 0]
